# baseline (speedup 1.0000x reference)
import functools

import jax
import jax.numpy as jnp
from jax import lax
from jax.experimental import pallas as pl
from jax.experimental.pallas import tpu as pltpu
from jax.experimental.pallas import tpu_sc as plsc

VOCAB = 100000
EMB = 128
CTX = 200
HIDDEN = 200

ROWS = 24
NCH = (HIDDEN + ROWS - 1) // ROWS

B_PER_W = 8
NW_USED = CTX // B_PER_W


def _sc_gather(emb_table, idx):
    info = plsc.get_sparse_core_info()
    ncores = info.num_cores

    mesh = plsc.VectorSubcoreMesh(core_axis_name="c", subcore_axis_name="s")

    @functools.partial(
        pl.kernel,
        out_type=jax.ShapeDtypeStruct((CTX, EMB), jnp.float32),
        mesh=mesh,
        scratch_types=[
            pltpu.VMEM((B_PER_W,), jnp.int32),
            pltpu.VMEM((B_PER_W, EMB), jnp.float32),
            pltpu.SemaphoreType.DMA,
        ],
    )
    def gather_kernel(table_hbm, idx_hbm, out_hbm, idx_v, rows_v, sem):
        wid = lax.axis_index("s") * ncores + lax.axis_index("c")

        @pl.when(wid < NW_USED)
        def _():
            base = wid * B_PER_W
            pltpu.sync_copy(idx_hbm.at[pl.ds(base, B_PER_W)], idx_v)
            pltpu.async_copy(table_hbm.at[idx_v], rows_v, sem).wait()
            pltpu.sync_copy(rows_v, out_hbm.at[pl.ds(base, B_PER_W)])

    return gather_kernel(emb_table, idx)


def _mlp_body(emb_ref, fcw_ref, fcb_ref, outwt_ref, outb_ref, out_ref,
              logits_ref, xt_ref):
    i = pl.program_id(0)

    @pl.when(i == 0)
    def _fc():
        x = lax.dot_general(
            emb_ref[...], fcw_ref[...],
            dimension_numbers=(((1,), (1,)), ((), ())),
            preferred_element_type=jnp.float32)
        xt_ref[...] = jnp.maximum(x + fcb_ref[...], 0.0)

    for jj in range(NCH):
        @pl.when(i == 1 + jj)
        def _proj(jj=jj):
            r = min(ROWS, HIDDEN - jj * ROWS)
            part = lax.dot_general(
                xt_ref[:, jj * ROWS:jj * ROWS + r], outwt_ref[:r, :],
                dimension_numbers=(((1,), (0,)), ((), ())),
                preferred_element_type=jnp.float32)
            if jj == 0:
                logits_ref[...] = part + outb_ref[...].reshape(1, VOCAB)
            else:
                logits_ref[...] = logits_ref[...] + part

    @pl.when(i == 1 + NCH)
    def _finish():
        lg = logits_ref[...]
        m = jnp.max(lg)
        s = jnp.sum(jnp.exp(lg - m))
        out_ref[...] = lg - (m + jnp.log(s))


def _mlp(embeds, fc_w, fc_b, out_wt, out_b):
    grid = (NCH + 2,)
    return pl.pallas_call(
        _mlp_body,
        grid=grid,
        in_specs=[
            pl.BlockSpec(memory_space=pltpu.MemorySpace.VMEM),
            pl.BlockSpec(memory_space=pltpu.MemorySpace.VMEM),
            pl.BlockSpec(memory_space=pltpu.MemorySpace.VMEM),
            pl.BlockSpec((ROWS, VOCAB), lambda i: (jnp.clip(i - 1, 0, NCH - 1), 0)),
            pl.BlockSpec(memory_space=pltpu.MemorySpace.VMEM),
        ],
        out_specs=pl.BlockSpec((1, VOCAB), lambda i: (0, 0)),
        out_shape=jax.ShapeDtypeStruct((1, VOCAB), jnp.float32),
        scratch_shapes=[
            pltpu.VMEM((1, VOCAB), jnp.float32),
            pltpu.VMEM((1, HIDDEN), jnp.float32),
        ],
    )(embeds, fc_w, fc_b, out_wt, out_b)


def kernel(input, emb_table, fc_w, fc_b, out_w, out_b):
    rows = _sc_gather(emb_table, input.astype(jnp.int32))
    embeds = rows.reshape(1, CTX * EMB)
    return _mlp(
        embeds,
        fc_w,
        fc_b.reshape(1, HIDDEN),
        out_w.T,
        out_b,
    )


# speedup vs baseline: 1.0192x; 1.0192x over previous
import functools

import jax
import jax.numpy as jnp
from jax import lax
from jax.experimental import pallas as pl
from jax.experimental.pallas import tpu as pltpu
from jax.experimental.pallas import tpu_sc as plsc

VOCAB = 100000
EMB = 128
CTX = 200
HIDDEN = 200

ROWS = 24
NCH = (HIDDEN + ROWS - 1) // ROWS

B_PER_W = 8
NW_USED = CTX // B_PER_W


def _sc_gather(emb_table, idx):
    info = plsc.get_sparse_core_info()
    ncores = info.num_cores

    mesh = plsc.VectorSubcoreMesh(core_axis_name="c", subcore_axis_name="s")

    @functools.partial(
        pl.kernel,
        out_type=jax.ShapeDtypeStruct((CTX, EMB), jnp.float32),
        mesh=mesh,
        scratch_types=[
            pltpu.VMEM((B_PER_W,), jnp.int32),
            pltpu.VMEM((B_PER_W, EMB), jnp.float32),
            pltpu.SemaphoreType.DMA,
        ],
    )
    def gather_kernel(table_hbm, idx_hbm, out_hbm, idx_v, rows_v, sem):
        wid = lax.axis_index("s") * ncores + lax.axis_index("c")

        @pl.when(wid < NW_USED)
        def _():
            base = wid * B_PER_W
            pltpu.sync_copy(idx_hbm.at[pl.ds(base, B_PER_W)], idx_v)
            pltpu.async_copy(table_hbm.at[idx_v], rows_v, sem).wait()
            pltpu.sync_copy(rows_v, out_hbm.at[pl.ds(base, B_PER_W)])

    return gather_kernel(emb_table, idx)


NBUF = 3


def _chunk_copy(outwt_ref, bufs_ref, sems, jj):
    r = min(ROWS, HIDDEN - jj * ROWS)
    return pltpu.make_async_copy(
        outwt_ref.at[pl.ds(jj * ROWS, r), :],
        bufs_ref.at[jj % NBUF, pl.ds(0, r), :],
        sems.at[jj % NBUF])


def _mlp_body(emb_ref, fcw_ref, fcb_ref, outwt_ref, outb_ref, out_ref,
              logits_ref, xt_ref, bufs_ref, sems):
    i = pl.program_id(0)

    @pl.when(i == 0)
    def _fc():
        for k in range(min(NBUF, NCH)):
            _chunk_copy(outwt_ref, bufs_ref, sems, k).start()
        x = lax.dot_general(
            emb_ref[...], fcw_ref[...],
            dimension_numbers=(((1,), (1,)), ((), ())),
            preferred_element_type=jnp.float32)
        xt_ref[...] = jnp.maximum(x + fcb_ref[...], 0.0)

    for jj in range(NCH):
        @pl.when(i == 1 + jj)
        def _proj(jj=jj):
            r = min(ROWS, HIDDEN - jj * ROWS)
            _chunk_copy(outwt_ref, bufs_ref, sems, jj).wait()
            part = lax.dot_general(
                xt_ref[:, jj * ROWS:jj * ROWS + r],
                bufs_ref[jj % NBUF, 0:r, :],
                dimension_numbers=(((1,), (0,)), ((), ())),
                preferred_element_type=jnp.float32)
            if jj == 0:
                logits_ref[...] = part + outb_ref[...].reshape(1, VOCAB)
            else:
                logits_ref[...] = logits_ref[...] + part
            if jj + NBUF < NCH:
                _chunk_copy(outwt_ref, bufs_ref, sems, jj + NBUF).start()

    @pl.when(i == 1 + NCH)
    def _finish():
        lg = logits_ref[...]
        m = jnp.max(lg)
        s = jnp.sum(jnp.exp(lg - m))
        out_ref[...] = lg - (m + jnp.log(s))


def _mlp(embeds, fc_w, fc_b, out_wt, out_b):
    grid = (NCH + 2,)
    return pl.pallas_call(
        _mlp_body,
        grid=grid,
        in_specs=[
            pl.BlockSpec(memory_space=pltpu.MemorySpace.VMEM),
            pl.BlockSpec(memory_space=pltpu.MemorySpace.VMEM),
            pl.BlockSpec(memory_space=pltpu.MemorySpace.VMEM),
            pl.BlockSpec(memory_space=pltpu.MemorySpace.HBM),
            pl.BlockSpec(memory_space=pltpu.MemorySpace.VMEM),
        ],
        out_specs=pl.BlockSpec((1, VOCAB), lambda i: (0, 0)),
        out_shape=jax.ShapeDtypeStruct((1, VOCAB), jnp.float32),
        scratch_shapes=[
            pltpu.VMEM((1, VOCAB), jnp.float32),
            pltpu.VMEM((1, HIDDEN), jnp.float32),
            pltpu.VMEM((NBUF, ROWS, VOCAB), jnp.float32),
            pltpu.SemaphoreType.DMA((NBUF,)),
        ],
    )(embeds, fc_w, fc_b, out_wt, out_b)


def kernel(input, emb_table, fc_w, fc_b, out_w, out_b):
    rows = _sc_gather(emb_table, input.astype(jnp.int32))
    embeds = rows.reshape(1, CTX * EMB)
    return _mlp(
        embeds,
        fc_w,
        fc_b.reshape(1, HIDDEN),
        out_w.T,
        out_b,
    )


# speedup vs baseline: 1.0215x; 1.0022x over previous
import functools

import jax
import jax.numpy as jnp
from jax import lax
from jax.experimental import pallas as pl
from jax.experimental.pallas import tpu as pltpu
from jax.experimental.pallas import tpu_sc as plsc

VOCAB = 100000
EMB = 128
CTX = 200
HIDDEN = 200

ROWS = 16
NCH = (HIDDEN + ROWS - 1) // ROWS

B_PER_W = 8
NW_USED = CTX // B_PER_W


def _sc_gather(emb_table, idx):
    info = plsc.get_sparse_core_info()
    ncores = info.num_cores

    mesh = plsc.VectorSubcoreMesh(core_axis_name="c", subcore_axis_name="s")

    @functools.partial(
        pl.kernel,
        out_type=jax.ShapeDtypeStruct((CTX, EMB), jnp.float32),
        mesh=mesh,
        scratch_types=[
            pltpu.VMEM((B_PER_W,), jnp.int32),
            pltpu.VMEM((B_PER_W, EMB), jnp.float32),
            pltpu.SemaphoreType.DMA,
        ],
    )
    def gather_kernel(table_hbm, idx_hbm, out_hbm, idx_v, rows_v, sem):
        wid = lax.axis_index("s") * ncores + lax.axis_index("c")

        @pl.when(wid < NW_USED)
        def _():
            base = wid * B_PER_W
            pltpu.sync_copy(idx_hbm.at[pl.ds(base, B_PER_W)], idx_v)
            pltpu.async_copy(table_hbm.at[idx_v], rows_v, sem).wait()
            pltpu.sync_copy(rows_v, out_hbm.at[pl.ds(base, B_PER_W)])

    return gather_kernel(emb_table, idx)


NBUF = 4


def _chunk_copy(outwt_ref, bufs_ref, sems, jj):
    r = min(ROWS, HIDDEN - jj * ROWS)
    return pltpu.make_async_copy(
        outwt_ref.at[pl.ds(jj * ROWS, r), :],
        bufs_ref.at[jj % NBUF, pl.ds(0, r), :],
        sems.at[jj % NBUF])


def _mlp_body(emb_ref, fcw_ref, fcb_ref, outwt_ref, outb_ref, out_ref,
              logits_ref, xt_ref, bufs_ref, sems):
    i = pl.program_id(0)

    @pl.when(i == 0)
    def _fc():
        for k in range(min(NBUF, NCH)):
            _chunk_copy(outwt_ref, bufs_ref, sems, k).start()
        x = lax.dot_general(
            emb_ref[...], fcw_ref[...],
            dimension_numbers=(((1,), (1,)), ((), ())),
            preferred_element_type=jnp.float32)
        xt_ref[...] = jnp.maximum(x + fcb_ref[...], 0.0)

    for jj in range(NCH):
        @pl.when(i == 1 + jj)
        def _proj(jj=jj):
            r = min(ROWS, HIDDEN - jj * ROWS)
            _chunk_copy(outwt_ref, bufs_ref, sems, jj).wait()
            part = lax.dot_general(
                xt_ref[:, jj * ROWS:jj * ROWS + r],
                bufs_ref[jj % NBUF, 0:r, :],
                dimension_numbers=(((1,), (0,)), ((), ())),
                preferred_element_type=jnp.float32)
            if jj == 0:
                logits_ref[...] = part + outb_ref[...].reshape(1, VOCAB)
            else:
                logits_ref[...] = logits_ref[...] + part
            if jj + NBUF < NCH:
                _chunk_copy(outwt_ref, bufs_ref, sems, jj + NBUF).start()

    @pl.when(i == 1 + NCH)
    def _finish():
        lg = logits_ref[...]
        m = jnp.max(lg)
        s = jnp.sum(jnp.exp(lg - m))
        out_ref[...] = lg - (m + jnp.log(s))


def _mlp(embeds, fc_w, fc_b, out_wt, out_b):
    grid = (NCH + 2,)
    return pl.pallas_call(
        _mlp_body,
        grid=grid,
        in_specs=[
            pl.BlockSpec(memory_space=pltpu.MemorySpace.VMEM),
            pl.BlockSpec(memory_space=pltpu.MemorySpace.VMEM),
            pl.BlockSpec(memory_space=pltpu.MemorySpace.VMEM),
            pl.BlockSpec(memory_space=pltpu.MemorySpace.HBM),
            pl.BlockSpec(memory_space=pltpu.MemorySpace.VMEM),
        ],
        out_specs=pl.BlockSpec((1, VOCAB), lambda i: (0, 0)),
        out_shape=jax.ShapeDtypeStruct((1, VOCAB), jnp.float32),
        scratch_shapes=[
            pltpu.VMEM((1, VOCAB), jnp.float32),
            pltpu.VMEM((1, HIDDEN), jnp.float32),
            pltpu.VMEM((NBUF, ROWS, VOCAB), jnp.float32),
            pltpu.SemaphoreType.DMA((NBUF,)),
        ],
    )(embeds, fc_w, fc_b, out_wt, out_b)


def kernel(input, emb_table, fc_w, fc_b, out_w, out_b):
    rows = _sc_gather(emb_table, input.astype(jnp.int32))
    embeds = rows.reshape(1, CTX * EMB)
    return _mlp(
        embeds,
        fc_w,
        fc_b.reshape(1, HIDDEN),
        out_w.T,
        out_b,
    )
